# Initial kernel scaffold; baseline (speedup 1.0000x reference)
import jax
import jax.numpy as jnp
from jax.experimental import pallas as pl


def kernel(value_feats, bool_feats, tweet_feats, des_feats, W1, b1, W2, b2, W3, b3, W4, b4, W_in, b_in, W_out, b_out, W_r, b_r):
    raise NotImplementedError("write your pallas kernel here")


# speedup vs baseline: 9.1346x; 9.1346x over previous
import jax
import jax.numpy as jnp
from jax.experimental import pallas as pl

_BLOCK = 2000


def _attr_block(vb_ref, tw_ref, de_ref, a12_ref, a3_ref, a4_ref, c_ref, o_ref):
    acc = jnp.dot(tw_ref[...], a3_ref[...], preferred_element_type=jnp.float32)
    acc = acc + jnp.dot(de_ref[...], a4_ref[...], preferred_element_type=jnp.float32)
    acc = acc + jnp.dot(vb_ref[...], a12_ref[...], preferred_element_type=jnp.float32)
    acc = acc + c_ref[...]
    o_ref[...] = jnp.where(acc >= 0.0, acc, 0.01 * acc)


def kernel(value_feats, bool_feats, tweet_feats, des_feats,
           W1, b1, W2, b2, W3, b3, W4, b4,
           W_in, b_in, W_out, b_out, W_r, b_r):
    N, VN = value_feats.shape
    BN = bool_feats.shape[1]
    TN = tweet_feats.shape[1]
    DN = des_feats.shape[1]
    FD = W_r.shape[0]
    E = W_out.shape[0]

    Wv = W_in[2 * E:3 * E]
    bv = b_in[2 * E:3 * E]
    m_t = W_r @ W_out @ Wv
    a1 = (m_t[:, 0 * FD:1 * FD] @ W1).T
    a2 = (m_t[:, 1 * FD:2 * FD] @ W2).T
    a3 = (m_t[:, 2 * FD:3 * FD] @ W3).T
    a4 = (m_t[:, 3 * FD:4 * FD] @ W4).T
    a12 = jnp.concatenate([a1, a2], axis=0)
    bx = jnp.concatenate([b1, b2, b3, b4])
    c = bx @ m_t.T + bv @ (W_r @ W_out).T + b_out @ W_r.T + b_r
    c2 = c.reshape(1, FD)
    vb = jnp.concatenate([value_feats, bool_feats], axis=1)

    grid = (pl.cdiv(N, _BLOCK),)
    out = pl.pallas_call(
        _attr_block,
        grid=grid,
        in_specs=[
            pl.BlockSpec((_BLOCK, VN + BN), lambda i: (i, 0)),
            pl.BlockSpec((_BLOCK, TN), lambda i: (i, 0)),
            pl.BlockSpec((_BLOCK, DN), lambda i: (i, 0)),
            pl.BlockSpec((VN + BN, FD), lambda i: (0, 0)),
            pl.BlockSpec((TN, FD), lambda i: (0, 0)),
            pl.BlockSpec((DN, FD), lambda i: (0, 0)),
            pl.BlockSpec((1, FD), lambda i: (0, 0)),
        ],
        out_specs=pl.BlockSpec((_BLOCK, FD), lambda i: (i, 0)),
        out_shape=jax.ShapeDtypeStruct((N, FD), jnp.float32),
    )(vb, tweet_feats, des_feats, a12, a3, a4, c2)
    return out
